# Initial kernel scaffold; baseline (speedup 1.0000x reference)
import jax
import jax.numpy as jnp
from jax.experimental import pallas as pl


def kernel(x, edge_index, W1, b1, W2, b2, W3, b3):
    raise NotImplementedError("write your pallas kernel here")


# speedup vs baseline: 32.2163x; 32.2163x over previous
import functools

import jax
import jax.numpy as jnp
from jax import lax
from jax.experimental import pallas as pl
from jax.experimental.pallas import tpu as pltpu
from jax.experimental.pallas import tpu_sc as plsc

N = 10000
E = 320000
D_IN = 128
H = 32
A = 3

CH = 128
NCH = E // CH
NC, NS = 2, 16
NW = NC * NS
ROWS = 1000
GRID = N // ROWS

_mesh = plsc.VectorSubcoreMesh(core_axis_name="c", subcore_axis_name="s")


@functools.partial(
    pl.kernel,
    out_type=jax.ShapeDtypeStruct((NC, N), jnp.float32),
    mesh=_mesh,
    scratch_types=[
        pltpu.VMEM((CH,), jnp.int32),
        pltpu.VMEM((CH,), jnp.float32),
        pltpu.VMEM_SHARED((N,), jnp.float32),
    ],
)
def _deg_kernel(edges, zeros_n, deg_out, dst_v, ones_v, deg_sh):
    cid = lax.axis_index("c")
    sid = lax.axis_index("s")
    wid = sid * NC + cid
    for i in range(CH // 16):
        ones_v[pl.ds(i * 16, 16)] = jnp.ones((16,), jnp.float32)

    @pl.when(sid == 0)
    def _():
        pltpu.sync_copy(zeros_n, deg_sh)

    plsc.subcore_barrier()
    nk = lax.shift_right_logical(NCH + NW - 1 - wid, 5)

    def body(k, carry):
        off = (wid + k * NW) * CH
        pltpu.sync_copy(edges.at[1, pl.ds(off, CH)], dst_v)
        pltpu.sync_copy(ones_v, deg_sh.at[dst_v], add=True)
        return carry

    lax.fori_loop(0, nk, body, 0)
    plsc.subcore_barrier()

    @pl.when(sid == 0)
    def _():
        pltpu.sync_copy(deg_sh, deg_out.at[cid])


@functools.partial(
    pl.kernel,
    out_type=(
        jax.ShapeDtypeStruct((NC, N, H), jnp.float32),
        jax.ShapeDtypeStruct((NC, N), jnp.float32),
    ),
    mesh=_mesh,
    scratch_types=[
        pltpu.VMEM((CH,), jnp.int32),
        pltpu.VMEM((CH,), jnp.int32),
        pltpu.VMEM((CH, H), jnp.float32),
        pltpu.VMEM((CH,), jnp.float32),
        pltpu.SemaphoreType.DMA,
        pltpu.SemaphoreType.DMA,
        pltpu.VMEM_SHARED((N, H), jnp.float32),
        pltpu.VMEM_SHARED((N,), jnp.float32),
    ],
    compiler_params=pltpu.CompilerParams(use_tc_tiling_on_sc=False),
)
def _agg_kernel(edges, g1, dinv, zeros_nh, zeros_n, agg_out, c_out,
                src_v, dst_v, rows_v, dvals_v, sem_r, sem_d, agg_sh, c_sh):
    cid = lax.axis_index("c")
    sid = lax.axis_index("s")
    wid = sid * NC + cid

    @pl.when(sid == 0)
    def _():
        pltpu.sync_copy(zeros_nh, agg_sh)
        pltpu.sync_copy(zeros_n, c_sh)

    plsc.subcore_barrier()
    nk = lax.shift_right_logical(NCH + NW - 1 - wid, 5)

    def body(k, carry):
        off = (wid + k * NW) * CH
        pltpu.sync_copy(edges.at[0, pl.ds(off, CH)], src_v)
        pltpu.sync_copy(edges.at[1, pl.ds(off, CH)], dst_v)
        cp_r = pltpu.async_copy(g1.at[src_v], rows_v, sem_r)
        cp_d = pltpu.async_copy(dinv.at[dst_v], dvals_v, sem_d)
        cp_r.wait()
        pltpu.sync_copy(rows_v, agg_sh.at[dst_v], add=True)
        cp_d.wait()
        pltpu.sync_copy(dvals_v, c_sh.at[src_v], add=True)
        return carry

    lax.fori_loop(0, nk, body, 0)
    plsc.subcore_barrier()

    @pl.when(sid == 0)
    def _():
        pltpu.sync_copy(agg_sh, agg_out.at[cid])
        pltpu.sync_copy(c_sh, c_out.at[cid])


def _dense1_body(x_ref, w1_ref, deg_ref, g1_ref, dinv_ref):
    deg = deg_ref[0] + deg_ref[1] + 1.0
    dinv = lax.rsqrt(deg)
    h = jnp.dot(x_ref[...], w1_ref[...], preferred_element_type=jnp.float32)
    g1_ref[...] = h * dinv
    dinv_ref[...] = dinv


def _dense1(x, W1, deg_p):
    return pl.pallas_call(
        _dense1_body,
        grid=(GRID,),
        in_specs=[
            pl.BlockSpec((ROWS, D_IN), lambda i: (i, 0)),
            pl.BlockSpec((D_IN, H), lambda i: (0, 0)),
            pl.BlockSpec((NC, ROWS, 1), lambda i: (0, i, 0)),
        ],
        out_specs=[
            pl.BlockSpec((ROWS, H), lambda i: (i, 0)),
            pl.BlockSpec((ROWS, 1), lambda i: (i, 0)),
        ],
        out_shape=[
            jax.ShapeDtypeStruct((N, H), jnp.float32),
            jax.ShapeDtypeStruct((N, 1), jnp.float32),
        ],
    )(x, W1, deg_p)


def _head_body(agg_ref, c_ref, g1_ref, dinv_ref, b1_ref, w2_ref, b2_ref,
               w3_ref, b3_ref, out_ref, acc):
    k = pl.program_id(0)

    @pl.when(k == 0)
    def _():
        acc[...] = jnp.zeros_like(acc)

    dinv = dinv_ref[...]
    agg = agg_ref[0] + agg_ref[1] + g1_ref[...]
    out1 = jnp.maximum(agg * dinv + b1_ref[...], 0.0)
    w = dinv * (c_ref[0] + c_ref[1] + dinv)
    acc[...] += jnp.sum(out1 * w, axis=0, keepdims=True)

    @pl.when(k == GRID - 1)
    def _():
        r = jnp.dot(acc[...], w2_ref[...],
                    preferred_element_type=jnp.float32) * (1.0 / N) + b2_ref[...]
        lg = jnp.dot(r, w3_ref[...],
                     preferred_element_type=jnp.float32) + b3_ref[...]
        m = jnp.max(lg, axis=1, keepdims=True)
        e = jnp.exp(lg - m)
        out_ref[...] = e / jnp.sum(e, axis=1, keepdims=True)


def _head(agg_p, c_p, g1, dinv, b1, W2, b2, W3, b3):
    return pl.pallas_call(
        _head_body,
        grid=(GRID,),
        in_specs=[
            pl.BlockSpec((NC, ROWS, H), lambda i: (0, i, 0)),
            pl.BlockSpec((NC, ROWS, 1), lambda i: (0, i, 0)),
            pl.BlockSpec((ROWS, H), lambda i: (i, 0)),
            pl.BlockSpec((ROWS, 1), lambda i: (i, 0)),
            pl.BlockSpec((1, H), lambda i: (0, 0)),
            pl.BlockSpec((H, H), lambda i: (0, 0)),
            pl.BlockSpec((1, H), lambda i: (0, 0)),
            pl.BlockSpec((H, A), lambda i: (0, 0)),
            pl.BlockSpec((1, A), lambda i: (0, 0)),
        ],
        out_specs=pl.BlockSpec((1, A), lambda i: (0, 0)),
        out_shape=jax.ShapeDtypeStruct((1, A), jnp.float32),
        scratch_shapes=[pltpu.VMEM((1, H), jnp.float32)],
    )(agg_p, c_p, g1, dinv, b1, W2, b2, W3, b3)


def kernel(x, edge_index, W1, b1, W2, b2, W3, b3):
    zeros_n = jnp.zeros((N,), jnp.float32)
    zeros_nh = jnp.zeros((N, H), jnp.float32)
    deg_p = _deg_kernel(edge_index, zeros_n)
    g1, dinv = _dense1(x, W1, deg_p.reshape(NC, N, 1))
    dinv1 = dinv.reshape(N)
    agg_p, c_p = _agg_kernel(edge_index, g1, dinv1, zeros_nh, zeros_n)
    out = _head(agg_p, c_p.reshape(NC, N, 1), g1, dinv,
                b1.reshape(1, H), W2, b2.reshape(1, H), W3, b3.reshape(1, A))
    return out


# speedup vs baseline: 38.4047x; 1.1921x over previous
import functools

import jax
import jax.numpy as jnp
from jax import lax
from jax.experimental import pallas as pl
from jax.experimental.pallas import tpu as pltpu
from jax.experimental.pallas import tpu_sc as plsc

N = 10000
E = 320000
D_IN = 128
H = 32
A = 3

CH = 128
NC, NS = 2, 16
NW = NC * NS
CPW = 80
NCH = NW * CPW
EP = NCH * CH
NP = 10240
ROWS = 1024
GRID = NP // ROWS
K = 16

_mesh = plsc.VectorSubcoreMesh(core_axis_name="c", subcore_axis_name="s")


@functools.partial(
    pl.kernel,
    out_type=jax.ShapeDtypeStruct((NC, NP), jnp.float32),
    mesh=_mesh,
    scratch_types=[
        pltpu.VMEM((CPW, CH), jnp.int32),
        pltpu.VMEM((CH,), jnp.float32),
        pltpu.SemaphoreType.DMA,
        pltpu.VMEM_SHARED((NP,), jnp.float32),
    ],
    compiler_params=pltpu.CompilerParams(use_tc_tiling_on_sc=False),
)
def _deg_kernel(edges, zeros_n, deg_out, didx, ones_v, sem_s, deg_sh):
    cid = lax.axis_index("c")
    sid = lax.axis_index("s")
    wid = sid * NC + cid
    for i in range(CH // 16):
        ones_v[pl.ds(i * 16, 16)] = jnp.ones((16,), jnp.float32)

    @pl.when(sid == 0)
    def _():
        pltpu.sync_copy(zeros_n, deg_sh)

    pltpu.sync_copy(edges.at[1, pl.ds(wid * CPW, CPW)], didx)
    plsc.subcore_barrier()

    def body(p, carry):
        descs = []
        for b in range(K):
            j = p * K + b
            descs.append(pltpu.async_copy(
                ones_v, deg_sh.at[didx.at[j]], sem_s, add=True))
        for d in descs:
            d.wait()
        return carry

    lax.fori_loop(0, CPW // K, body, 0)
    plsc.subcore_barrier()

    @pl.when(sid == 0)
    def _():
        pltpu.sync_copy(deg_sh, deg_out.at[cid])


@functools.partial(
    pl.kernel,
    out_type=(
        jax.ShapeDtypeStruct((NC, NP, H), jnp.float32),
        jax.ShapeDtypeStruct((NC, NP), jnp.float32),
    ),
    mesh=_mesh,
    scratch_types=[
        pltpu.VMEM((CPW, CH), jnp.int32),
        pltpu.VMEM((CPW, CH), jnp.int32),
        pltpu.VMEM((K, CH, H), jnp.float32),
        pltpu.VMEM((K, CH), jnp.float32),
        pltpu.SemaphoreType.DMA,
        pltpu.SemaphoreType.DMA,
        pltpu.SemaphoreType.DMA,
        pltpu.SemaphoreType.DMA,
        pltpu.VMEM_SHARED((NP, H), jnp.float32),
        pltpu.VMEM_SHARED((NP,), jnp.float32),
    ],
    compiler_params=pltpu.CompilerParams(use_tc_tiling_on_sc=False),
)
def _agg_kernel(edges, g1, dinv, zeros_nh, zeros_n, agg_out, c_out,
                sidx, didx, rows, dvals, sem_g, sem_d, sem_s, sem_c,
                agg_sh, c_sh):
    cid = lax.axis_index("c")
    sid = lax.axis_index("s")
    wid = sid * NC + cid

    @pl.when(sid == 0)
    def _():
        pltpu.sync_copy(zeros_nh, agg_sh)
        pltpu.sync_copy(zeros_n, c_sh)

    pltpu.sync_copy(edges.at[0, pl.ds(wid * CPW, CPW)], sidx)
    pltpu.sync_copy(edges.at[1, pl.ds(wid * CPW, CPW)], didx)
    plsc.subcore_barrier()

    def body(p, carry):
        dg, dd = [], []
        for b in range(K):
            j = p * K + b
            dg.append(pltpu.async_copy(g1.at[sidx.at[j]], rows.at[b], sem_g))
            dd.append(pltpu.async_copy(dinv.at[didx.at[j]], dvals.at[b], sem_d))
        ds = []
        for b in range(K):
            j = p * K + b
            dg[b].wait()
            ds.append(pltpu.async_copy(
                rows.at[b], agg_sh.at[didx.at[j]], sem_s, add=True))
            dd[b].wait()
            ds.append(pltpu.async_copy(
                dvals.at[b], c_sh.at[sidx.at[j]], sem_c, add=True))
        for d in ds:
            d.wait()
        return carry

    lax.fori_loop(0, CPW // K, body, 0)
    plsc.subcore_barrier()

    @pl.when(sid == 0)
    def _():
        pltpu.sync_copy(agg_sh, agg_out.at[cid])
        pltpu.sync_copy(c_sh, c_out.at[cid])


def _dense1_body(x_ref, w1_ref, deg_ref, g1_ref, dinv_ref):
    deg = deg_ref[0] + deg_ref[1] + 1.0
    dinv = lax.rsqrt(deg)
    h = jnp.dot(x_ref[...], w1_ref[...], preferred_element_type=jnp.float32)
    g1_ref[...] = h * dinv
    dinv_ref[...] = dinv


def _dense1(x, W1, deg_p):
    return pl.pallas_call(
        _dense1_body,
        grid=(GRID,),
        in_specs=[
            pl.BlockSpec((ROWS, D_IN), lambda i: (i, 0)),
            pl.BlockSpec((D_IN, H), lambda i: (0, 0)),
            pl.BlockSpec((NC, ROWS, 1), lambda i: (0, i, 0)),
        ],
        out_specs=[
            pl.BlockSpec((ROWS, H), lambda i: (i, 0)),
            pl.BlockSpec((ROWS, 1), lambda i: (i, 0)),
        ],
        out_shape=[
            jax.ShapeDtypeStruct((NP, H), jnp.float32),
            jax.ShapeDtypeStruct((NP, 1), jnp.float32),
        ],
    )(x, W1, deg_p)


def _head_body(agg_ref, c_ref, g1_ref, dinv_ref, b1_ref, w2_ref, b2_ref,
               w3_ref, b3_ref, out_ref, acc):
    k = pl.program_id(0)

    @pl.when(k == 0)
    def _():
        acc[...] = jnp.zeros_like(acc)

    dinv = dinv_ref[...]
    agg = agg_ref[0] + agg_ref[1] + g1_ref[...]
    out1 = jnp.maximum(agg * dinv + b1_ref[...], 0.0)
    w = dinv * (c_ref[0] + c_ref[1] + dinv)
    row = k * ROWS + lax.broadcasted_iota(jnp.int32, (ROWS, 1), 0)
    w = jnp.where(row < N, w, 0.0)
    acc[...] += jnp.sum(out1 * w, axis=0, keepdims=True)

    @pl.when(k == GRID - 1)
    def _():
        r = jnp.dot(acc[...], w2_ref[...],
                    preferred_element_type=jnp.float32) * (1.0 / N) + b2_ref[...]
        lg = jnp.dot(r, w3_ref[...],
                     preferred_element_type=jnp.float32) + b3_ref[...]
        m = jnp.max(lg, axis=1, keepdims=True)
        e = jnp.exp(lg - m)
        out_ref[...] = e / jnp.sum(e, axis=1, keepdims=True)


def _head(agg_p, c_p, g1, dinv, b1, W2, b2, W3, b3):
    return pl.pallas_call(
        _head_body,
        grid=(GRID,),
        in_specs=[
            pl.BlockSpec((NC, ROWS, H), lambda i: (0, i, 0)),
            pl.BlockSpec((NC, ROWS, 1), lambda i: (0, i, 0)),
            pl.BlockSpec((ROWS, H), lambda i: (i, 0)),
            pl.BlockSpec((ROWS, 1), lambda i: (i, 0)),
            pl.BlockSpec((1, H), lambda i: (0, 0)),
            pl.BlockSpec((H, H), lambda i: (0, 0)),
            pl.BlockSpec((1, H), lambda i: (0, 0)),
            pl.BlockSpec((H, A), lambda i: (0, 0)),
            pl.BlockSpec((1, A), lambda i: (0, 0)),
        ],
        out_specs=pl.BlockSpec((1, A), lambda i: (0, 0)),
        out_shape=jax.ShapeDtypeStruct((1, A), jnp.float32),
        scratch_shapes=[pltpu.VMEM((1, H), jnp.float32)],
    )(agg_p, c_p, g1, dinv, b1, W2, b2, W3, b3)


def kernel(x, edge_index, W1, b1, W2, b2, W3, b3):
    pad = jnp.full((2, EP - E), N, jnp.int32)
    edges = jnp.concatenate([edge_index, pad], axis=1).reshape(2, NCH, CH)
    x_pad = jnp.pad(x, ((0, NP - N), (0, 0)))
    zeros_n = jnp.zeros((NP,), jnp.float32)
    zeros_nh = jnp.zeros((NP, H), jnp.float32)
    deg_p = _deg_kernel(edges, zeros_n)
    g1, dinv = _dense1(x_pad, W1, deg_p.reshape(NC, NP, 1))
    agg_p, c_p = _agg_kernel(edges, g1, dinv.reshape(NP), zeros_nh, zeros_n)
    out = _head(agg_p, c_p.reshape(NC, NP, 1), g1, dinv,
                b1.reshape(1, H), W2, b2.reshape(1, H), W3, b3.reshape(1, A))
    return out


# speedup vs baseline: 56.6221x; 1.4744x over previous
import functools

import jax
import jax.numpy as jnp
from jax import lax
from jax.experimental import pallas as pl
from jax.experimental.pallas import tpu as pltpu
from jax.experimental.pallas import tpu_sc as plsc

N = 10000
E = 320000
D_IN = 128
H = 32
A = 3

CH = 128
NC, NS = 2, 16
NW = NC * NS
CPW = 80
NCH = NW * CPW
EP = NCH * CH
NP = 10240
ROWS = 1024
GRID = NP // ROWS
K = 16

_mesh = plsc.VectorSubcoreMesh(core_axis_name="c", subcore_axis_name="s")


@functools.partial(
    pl.kernel,
    out_type=jax.ShapeDtypeStruct((NC, NP), jnp.float32),
    mesh=_mesh,
    scratch_types=[
        pltpu.VMEM((CPW, CH), jnp.int32),
        pltpu.VMEM((CH,), jnp.float32),
        pltpu.SemaphoreType.DMA,
        pltpu.VMEM_SHARED((NP,), jnp.float32),
    ],
    compiler_params=pltpu.CompilerParams(use_tc_tiling_on_sc=False),
)
def _deg_kernel(edges, zeros_n, deg_out, didx, ones_v, sem_s, deg_sh):
    cid = lax.axis_index("c")
    sid = lax.axis_index("s")
    wid = sid * NC + cid
    for i in range(CH // 16):
        ones_v[pl.ds(i * 16, 16)] = jnp.ones((16,), jnp.float32)

    @pl.when(sid == 0)
    def _():
        pltpu.sync_copy(zeros_n, deg_sh)

    pltpu.sync_copy(edges.at[1, pl.ds(wid * CPW, CPW)], didx)
    plsc.subcore_barrier()

    def body(p, carry):
        descs = []
        for b in range(K):
            j = p * K + b
            descs.append(pltpu.async_copy(
                ones_v, deg_sh.at[didx.at[j]], sem_s, add=True))
        for d in descs:
            d.wait()
        return carry

    lax.fori_loop(0, CPW // K, body, 0)
    plsc.subcore_barrier()

    @pl.when(sid == 0)
    def _():
        pltpu.sync_copy(deg_sh, deg_out.at[cid])


@functools.partial(
    pl.kernel,
    out_type=(
        jax.ShapeDtypeStruct((NC, NP, H), jnp.float32),
        jax.ShapeDtypeStruct((NC, NP), jnp.float32),
    ),
    mesh=_mesh,
    scratch_types=[
        pltpu.VMEM((CPW, CH), jnp.int32),
        pltpu.VMEM((CPW, CH), jnp.int32),
        pltpu.VMEM((K, CH, H), jnp.float32),
        pltpu.VMEM((K, CH), jnp.float32),
        pltpu.SemaphoreType.DMA,
        pltpu.SemaphoreType.DMA,
        pltpu.SemaphoreType.DMA,
        pltpu.SemaphoreType.DMA,
        pltpu.VMEM_SHARED((NP, H), jnp.float32),
        pltpu.VMEM_SHARED((NP,), jnp.float32),
    ],
    compiler_params=pltpu.CompilerParams(use_tc_tiling_on_sc=False),
)
def _agg_kernel(edges, g1, dinv, zeros_nh, zeros_n, agg_out, c_out,
                sidx, didx, rows, dvals, sem_g, sem_d, sem_s, sem_c,
                agg_sh, c_sh):
    cid = lax.axis_index("c")
    sid = lax.axis_index("s")
    wid = sid * NC + cid

    @pl.when(sid == 0)
    def _():
        pltpu.sync_copy(zeros_nh, agg_sh)
        pltpu.sync_copy(zeros_n, c_sh)

    pltpu.sync_copy(edges.at[0, pl.ds(wid * CPW, CPW)], sidx)
    pltpu.sync_copy(edges.at[1, pl.ds(wid * CPW, CPW)], didx)
    plsc.subcore_barrier()

    def body(p, carry):
        dg, dd = [], []
        for b in range(K):
            j = p * K + b
            dg.append(pltpu.async_copy(g1.at[sidx.at[j]], rows.at[b], sem_g))
            dd.append(pltpu.async_copy(dinv.at[didx.at[j]], dvals.at[b], sem_d))
        ds = []
        for b in range(K):
            j = p * K + b
            dg[b].wait()
            ds.append(pltpu.async_copy(
                rows.at[b], agg_sh.at[didx.at[j]], sem_s, add=True))
            dd[b].wait()
            ds.append(pltpu.async_copy(
                dvals.at[b], c_sh.at[sidx.at[j]], sem_c, add=True))
        for d in ds:
            d.wait()
        return carry

    lax.fori_loop(0, CPW // K, body, 0)
    plsc.subcore_barrier()

    @pl.when(sid == 0)
    def _():
        pltpu.sync_copy(agg_sh, agg_out.at[cid])
        pltpu.sync_copy(c_sh, c_out.at[cid])


def _dense1_body(x_ref, w1_ref, deg_ref, g1_ref, dinv_ref):
    deg = deg_ref[0] + deg_ref[1] + 1.0
    dinv = lax.rsqrt(deg)
    h = jnp.dot(x_ref[...], w1_ref[...], preferred_element_type=jnp.float32)
    g1_ref[...] = h * dinv
    dinv_ref[...] = dinv


def _dense1(x, W1, deg_p):
    return pl.pallas_call(
        _dense1_body,
        grid=(GRID,),
        in_specs=[
            pl.BlockSpec((ROWS, D_IN), lambda i: (i, 0)),
            pl.BlockSpec((D_IN, H), lambda i: (0, 0)),
            pl.BlockSpec((NC, ROWS, 1), lambda i: (0, i, 0)),
        ],
        out_specs=[
            pl.BlockSpec((ROWS, H), lambda i: (i, 0)),
            pl.BlockSpec((ROWS, 1), lambda i: (i, 0)),
        ],
        out_shape=[
            jax.ShapeDtypeStruct((NP, H), jnp.float32),
            jax.ShapeDtypeStruct((NP, 1), jnp.float32),
        ],
    )(x, W1, deg_p)


def _head_body(agg_ref, c_ref, g1_ref, dinv_ref, b1_ref, w2_ref, b2_ref,
               w3_ref, b3_ref, out_ref, acc):
    k = pl.program_id(0)

    @pl.when(k == 0)
    def _():
        acc[...] = jnp.zeros_like(acc)

    dinv = dinv_ref[...]
    agg = agg_ref[0] + agg_ref[1] + g1_ref[...]
    out1 = jnp.maximum(agg * dinv + b1_ref[...], 0.0)
    w = dinv * (c_ref[0] + c_ref[1] + dinv)
    row = k * ROWS + lax.broadcasted_iota(jnp.int32, (ROWS, 1), 0)
    w = jnp.where(row < N, w, 0.0)
    acc[...] += jnp.sum(out1 * w, axis=0, keepdims=True)

    @pl.when(k == GRID - 1)
    def _():
        r = jnp.dot(acc[...], w2_ref[...],
                    preferred_element_type=jnp.float32) * (1.0 / N) + b2_ref[...]
        lg = jnp.dot(r, w3_ref[...],
                     preferred_element_type=jnp.float32) + b3_ref[...]
        m = jnp.max(lg, axis=1, keepdims=True)
        e = jnp.exp(lg - m)
        out_ref[...] = e / jnp.sum(e, axis=1, keepdims=True)


def _head(agg_p, c_p, g1, dinv, b1, W2, b2, W3, b3):
    return pl.pallas_call(
        _head_body,
        grid=(GRID,),
        in_specs=[
            pl.BlockSpec((NC, ROWS, H), lambda i: (0, i, 0)),
            pl.BlockSpec((NC, ROWS, 1), lambda i: (0, i, 0)),
            pl.BlockSpec((ROWS, H), lambda i: (i, 0)),
            pl.BlockSpec((ROWS, 1), lambda i: (i, 0)),
            pl.BlockSpec((1, H), lambda i: (0, 0)),
            pl.BlockSpec((H, H), lambda i: (0, 0)),
            pl.BlockSpec((1, H), lambda i: (0, 0)),
            pl.BlockSpec((H, A), lambda i: (0, 0)),
            pl.BlockSpec((1, A), lambda i: (0, 0)),
        ],
        out_specs=pl.BlockSpec((1, A), lambda i: (0, 0)),
        out_shape=jax.ShapeDtypeStruct((1, A), jnp.float32),
        scratch_shapes=[pltpu.VMEM((1, H), jnp.float32)],
    )(agg_p, c_p, g1, dinv, b1, W2, b2, W3, b3)


def kernel(x, edge_index, W1, b1, W2, b2, W3, b3):
    pad = N + jnp.tile(jnp.arange(EP - E, dtype=jnp.int32) % (NP - N), (2, 1))
    edges = jnp.concatenate([edge_index, pad], axis=1).reshape(2, NCH, CH)
    x_pad = jnp.pad(x, ((0, NP - N), (0, 0)))
    zeros_n = jnp.zeros((NP,), jnp.float32)
    zeros_nh = jnp.zeros((NP, H), jnp.float32)
    deg_p = _deg_kernel(edges, zeros_n)
    g1, dinv = _dense1(x_pad, W1, deg_p.reshape(NC, NP, 1))
    agg_p, c_p = _agg_kernel(edges, g1, dinv.reshape(NP), zeros_nh, zeros_n)
    out = _head(agg_p, c_p.reshape(NC, NP, 1), g1, dinv,
                b1.reshape(1, H), W2, b2.reshape(1, H), W3, b3.reshape(1, A))
    return out
